# Initial kernel scaffold; baseline (speedup 1.0000x reference)
import jax
import jax.numpy as jnp
from jax.experimental import pallas as pl


def kernel(atomic_energies, scale, shift, atomic_numbers):
    raise NotImplementedError("write your pallas kernel here")


# speedup vs baseline: 426.0932x; 426.0932x over previous
import functools

import jax
import jax.numpy as jnp
from jax import lax
from jax.experimental import pallas as pl
from jax.experimental.pallas import tpu as pltpu
from jax.experimental.pallas import tpu_sc as plsc

N_ATOMS = 2_000_000
N_TABLE = 95

NC = 2
NS = 16
NW = NC * NS
LANES = 16

SPAN = 62_496
CHUNK = 10_416
N_CHUNKS = SPAN // CHUNK
TAIL = N_ATOMS - NW * SPAN
TAIL_BASE = NW * SPAN


def _sc_body(e_hbm, scale_hbm, shift_hbm, z_hbm, out_hbm,
             scale_v, shift_v, z_v, e_v, o_v):
    wid = lax.axis_index("s") * NC + lax.axis_index("c")
    base = wid * SPAN

    pltpu.sync_copy(scale_hbm, scale_v)
    pltpu.sync_copy(shift_hbm, shift_v)

    def compute(n_elems):
        def body(i, _):
            off = i * LANES
            idx = z_v[pl.ds(off, LANES)]
            s = plsc.load_gather(scale_v, [idx])
            t = plsc.load_gather(shift_v, [idx])
            e = e_v[pl.ds(off, LANES)]
            o_v[pl.ds(off, LANES)] = e * s + t
            return 0
        lax.fori_loop(0, n_elems // LANES, body, 0)

    for c in range(N_CHUNKS):
        off = base + c * CHUNK
        pltpu.sync_copy(z_hbm.at[pl.ds(off, CHUNK)], z_v)
        pltpu.sync_copy(e_hbm.at[pl.ds(off, CHUNK)], e_v)
        compute(CHUNK)
        pltpu.sync_copy(o_v, out_hbm.at[pl.ds(off, CHUNK)])

    @pl.when(wid == 0)
    def _():
        pltpu.sync_copy(z_hbm.at[pl.ds(TAIL_BASE, TAIL)], z_v.at[pl.ds(0, TAIL)])
        pltpu.sync_copy(e_hbm.at[pl.ds(TAIL_BASE, TAIL)], e_v.at[pl.ds(0, TAIL)])
        compute(TAIL)
        pltpu.sync_copy(o_v.at[pl.ds(0, TAIL)], out_hbm.at[pl.ds(TAIL_BASE, TAIL)])


@jax.jit
def _atom_scaling_sc(atomic_energies, scale, shift, atomic_numbers):
    mesh = plsc.VectorSubcoreMesh(core_axis_name="c", subcore_axis_name="s")
    return pl.kernel(
        _sc_body,
        out_type=jax.ShapeDtypeStruct((N_ATOMS,), jnp.float32),
        mesh=mesh,
        compiler_params=pltpu.CompilerParams(needs_layout_passes=False),
        scratch_types=[
            pltpu.VMEM((N_TABLE,), jnp.float32),
            pltpu.VMEM((N_TABLE,), jnp.float32),
            pltpu.VMEM((CHUNK,), jnp.int32),
            pltpu.VMEM((CHUNK,), jnp.float32),
            pltpu.VMEM((CHUNK,), jnp.float32),
        ],
    )(atomic_energies, scale, shift, atomic_numbers)


def kernel(atomic_energies, scale, shift, atomic_numbers):
    return _atom_scaling_sc(atomic_energies, scale, shift,
                            atomic_numbers.astype(jnp.int32))


# speedup vs baseline: 757.6718x; 1.7782x over previous
import jax
import jax.numpy as jnp
from jax import lax
from jax.experimental import pallas as pl
from jax.experimental.pallas import tpu as pltpu
from jax.experimental.pallas import tpu_sc as plsc

N_ATOMS = 2_000_000
N_TABLE = 95

NC = 2
NS = 16
NW = NC * NS
LANES = 16

SPAN = 62_496
CHUNK = 10_416
N_CHUNKS = SPAN // CHUNK
TAIL = N_ATOMS - NW * SPAN
TAIL_BASE = NW * SPAN


def _sc_body(e_hbm, scale_hbm, shift_hbm, z_hbm, out_hbm,
             scale_v, shift_v, z0, z1, e0, e1, o0, o1,
             isem0, isem1, osem0, osem1):
    wid = lax.axis_index("s") * NC + lax.axis_index("c")
    base = wid * SPAN

    zb, eb, ob = [z0, z1], [e0, e1], [o0, o1]
    isem, osem = [isem0, isem1], [osem0, osem1]

    pltpu.sync_copy(scale_hbm, scale_v)
    pltpu.sync_copy(shift_hbm, shift_v)

    def compute(n_elems, z_v, e_v, o_v):
        @plsc.parallel_loop(0, n_elems, step=LANES, unroll=4)
        def _(off):
            idx = z_v[pl.ds(off, LANES)]
            s = plsc.load_gather(scale_v, [idx])
            t = plsc.load_gather(shift_v, [idx])
            e = e_v[pl.ds(off, LANES)]
            o_v[pl.ds(off, LANES)] = e * s + t

    in_handles, out_handles = {}, {}

    def start_in(c):
        b = c % 2
        off = base + c * CHUNK
        in_handles[c] = (
            pltpu.async_copy(z_hbm.at[pl.ds(off, CHUNK)], zb[b], isem[b]),
            pltpu.async_copy(e_hbm.at[pl.ds(off, CHUNK)], eb[b], isem[b]),
        )

    start_in(0)
    for c in range(N_CHUNKS):
        b = c % 2
        if c + 1 < N_CHUNKS:
            start_in(c + 1)
        for h in in_handles.pop(c):
            h.wait()
        if c - 2 >= 0:
            out_handles.pop(c - 2).wait()
        compute(CHUNK, zb[b], eb[b], ob[b])
        out_handles[c] = pltpu.async_copy(
            ob[b], out_hbm.at[pl.ds(base + c * CHUNK, CHUNK)], osem[b])

    for c in sorted(out_handles):
        out_handles.pop(c).wait()

    @pl.when(wid == 0)
    def _():
        pltpu.sync_copy(z_hbm.at[pl.ds(TAIL_BASE, TAIL)], z0.at[pl.ds(0, TAIL)])
        pltpu.sync_copy(e_hbm.at[pl.ds(TAIL_BASE, TAIL)], e0.at[pl.ds(0, TAIL)])
        compute(TAIL, z0, e0, o0)
        pltpu.sync_copy(o0.at[pl.ds(0, TAIL)], out_hbm.at[pl.ds(TAIL_BASE, TAIL)])


@jax.jit
def _atom_scaling_sc(atomic_energies, scale, shift, atomic_numbers):
    mesh = plsc.VectorSubcoreMesh(core_axis_name="c", subcore_axis_name="s")
    return pl.kernel(
        _sc_body,
        out_type=jax.ShapeDtypeStruct((N_ATOMS,), jnp.float32),
        mesh=mesh,
        compiler_params=pltpu.CompilerParams(needs_layout_passes=False),
        scratch_types=[
            pltpu.VMEM((N_TABLE,), jnp.float32),
            pltpu.VMEM((N_TABLE,), jnp.float32),
            pltpu.VMEM((CHUNK,), jnp.int32),
            pltpu.VMEM((CHUNK,), jnp.int32),
            pltpu.VMEM((CHUNK,), jnp.float32),
            pltpu.VMEM((CHUNK,), jnp.float32),
            pltpu.VMEM((CHUNK,), jnp.float32),
            pltpu.VMEM((CHUNK,), jnp.float32),
            pltpu.SemaphoreType.DMA,
            pltpu.SemaphoreType.DMA,
            pltpu.SemaphoreType.DMA,
            pltpu.SemaphoreType.DMA,
        ],
    )(atomic_energies, scale, shift, atomic_numbers)


def kernel(atomic_energies, scale, shift, atomic_numbers):
    return _atom_scaling_sc(atomic_energies, scale, shift,
                            atomic_numbers.astype(jnp.int32))
